# Initial kernel scaffold; baseline (speedup 1.0000x reference)
import jax
import jax.numpy as jnp
from jax.experimental import pallas as pl


def kernel(x, W_in, b_in, tod_table, dow_table, adp):
    raise NotImplementedError("write your pallas kernel here")


# speedup vs baseline: 1.3261x; 1.3261x over previous
import functools
import jax
import jax.numpy as jnp
from jax import lax
from jax.experimental import pallas as pl
from jax.experimental.pallas import tpu as pltpu


def _embed_body(x_ref, w_ref, b_ref, tod_ref, dow_ref, adp_ref, out_ref, *, steps_per_day):
    xv = x_ref[0]
    x1 = xv[:, 1:2]
    x2 = xv[:, 2:3]
    w = w_ref[...]
    xp = (
        xv[:, 0:1] * w[0:1, :]
        + x1 * w[1:2, :]
        + x2 * w[2:3, :]
        + b_ref[...]
    )

    n_tod = tod_ref.shape[0]
    ti = (x1 * jnp.float32(steps_per_day)).astype(jnp.int32)
    oh_t = (ti == lax.broadcasted_iota(jnp.int32, (1, n_tod), 1)).astype(jnp.float32)
    tod_emb = jnp.dot(oh_t, tod_ref[...], preferred_element_type=jnp.float32)

    n_dow = dow_ref.shape[0]
    di = x2.astype(jnp.int32)
    oh_d = (di == lax.broadcasted_iota(jnp.int32, (1, n_dow), 1)).astype(jnp.float32)
    dow_emb = jnp.dot(oh_d, dow_ref[...], preferred_element_type=jnp.float32)

    out_ref[0] = jnp.concatenate([xp, tod_emb, dow_emb, adp_ref[...]], axis=-1)


def kernel(x, W_in, b_in, tod_table, dow_table, adp):
    B, L, N, D = x.shape
    E = W_in.shape[1]
    A = adp.shape[-1]
    R = L * N
    OUT = E * 3 + A
    CHUNK = 512
    assert R % CHUNK == 0

    x3 = x.reshape(B, R, D)
    adp2 = adp.reshape(R, A)
    b2 = b_in.reshape(1, E)

    grid = (B, R // CHUNK)
    out = pl.pallas_call(
        functools.partial(_embed_body, steps_per_day=288),
        grid=grid,
        in_specs=[
            pl.BlockSpec((1, CHUNK, D), lambda b, c: (b, c, 0)),
            pl.BlockSpec((D, E), lambda b, c: (0, 0)),
            pl.BlockSpec((1, E), lambda b, c: (0, 0)),
            pl.BlockSpec(tod_table.shape, lambda b, c: (0, 0)),
            pl.BlockSpec(dow_table.shape, lambda b, c: (0, 0)),
            pl.BlockSpec((CHUNK, A), lambda b, c: (c, 0)),
        ],
        out_specs=pl.BlockSpec((1, CHUNK, OUT), lambda b, c: (b, c, 0)),
        out_shape=jax.ShapeDtypeStruct((B, R, OUT), jnp.float32),
    )(x3, W_in, b2, tod_table, dow_table, adp2)
    return out.reshape(B, L, N, OUT)


# speedup vs baseline: 1.5137x; 1.1415x over previous
import functools
import jax
import jax.numpy as jnp
from jax import lax
from jax.experimental import pallas as pl
from jax.experimental.pallas import tpu as pltpu


def _embed_body(x_ref, w_ref, b_ref, tod_ref, dow_ref, adp_ref, out_ref, *, steps_per_day):
    xv = x_ref[0]
    x1 = xv[:, 1:2]
    x2 = xv[:, 2:3]
    w = w_ref[...]
    xp = (
        xv[:, 0:1] * w[0:1, :]
        + x1 * w[1:2, :]
        + x2 * w[2:3, :]
        + b_ref[...]
    )

    n_tod = tod_ref.shape[0]
    ti = (x1 * jnp.float32(steps_per_day)).astype(jnp.int32)
    oh_t = (ti == lax.broadcasted_iota(jnp.int32, (1, n_tod), 1)).astype(jnp.float32)
    tod_emb = jnp.dot(oh_t, tod_ref[...], preferred_element_type=jnp.float32)

    n_dow = dow_ref.shape[0]
    di = x2.astype(jnp.int32)
    oh_d = (di == lax.broadcasted_iota(jnp.int32, (1, n_dow), 1)).astype(jnp.float32)
    dow_emb = jnp.dot(oh_d, dow_ref[...], preferred_element_type=jnp.float32)

    out_ref[0] = jnp.concatenate([xp, tod_emb, dow_emb, adp_ref[...]], axis=-1)


def kernel(x, W_in, b_in, tod_table, dow_table, adp):
    B, L, N, D = x.shape
    E = W_in.shape[1]
    A = adp.shape[-1]
    R = L * N
    OUT = E * 3 + A
    CHUNK = 1024
    assert R % CHUNK == 0

    x3 = x.reshape(B, R, D)
    adp2 = adp.reshape(R, A)
    b2 = b_in.reshape(1, E)

    grid = (R // CHUNK, B)
    out = pl.pallas_call(
        functools.partial(_embed_body, steps_per_day=288),
        grid=grid,
        in_specs=[
            pl.BlockSpec((1, CHUNK, D), lambda c, b: (b, c, 0)),
            pl.BlockSpec((D, E), lambda c, b: (0, 0)),
            pl.BlockSpec((1, E), lambda c, b: (0, 0)),
            pl.BlockSpec(tod_table.shape, lambda c, b: (0, 0)),
            pl.BlockSpec(dow_table.shape, lambda c, b: (0, 0)),
            pl.BlockSpec((CHUNK, A), lambda c, b: (c, 0)),
        ],
        out_specs=pl.BlockSpec((1, CHUNK, OUT), lambda c, b: (b, c, 0)),
        out_shape=jax.ShapeDtypeStruct((B, R, OUT), jnp.float32),
    )(x3, W_in, b2, tod_table, dow_table, adp2)
    return out.reshape(B, L, N, OUT)


# speedup vs baseline: 3.3900x; 2.2396x over previous
import functools
import jax
import jax.numpy as jnp
from jax import lax
from jax.experimental import pallas as pl
from jax.experimental.pallas import tpu as pltpu


def _embed_body(x_ref, w_ref, b_ref, tod_ref, dow_ref, adp_ref, out_ref, *, steps_per_day):
    xv = x_ref[0, 0]
    x1 = xv[:, 1:2]
    x2 = xv[:, 2:3]
    w = w_ref[...]
    xp = (
        xv[:, 0:1] * w[0:1, :]
        + x1 * w[1:2, :]
        + x2 * w[2:3, :]
        + b_ref[...]
    )

    n_tod = tod_ref.shape[0]
    ti = (x1 * jnp.float32(steps_per_day)).astype(jnp.int32)
    oh_t = (ti == lax.broadcasted_iota(jnp.int32, (1, n_tod), 1)).astype(jnp.float32)
    tod_emb = jnp.dot(oh_t, tod_ref[...], preferred_element_type=jnp.float32)

    n_dow = dow_ref.shape[0]
    di = x2.astype(jnp.int32)
    oh_d = (di == lax.broadcasted_iota(jnp.int32, (1, n_dow), 1)).astype(jnp.float32)
    dow_emb = jnp.dot(oh_d, dow_ref[...], preferred_element_type=jnp.float32)

    out_ref[0, 0] = jnp.concatenate([xp, tod_emb, dow_emb, adp_ref[0]], axis=-1)


def kernel(x, W_in, b_in, tod_table, dow_table, adp):
    B, L, N, D = x.shape
    E = W_in.shape[1]
    A = adp.shape[-1]
    OUT = E * 3 + A
    CHUNK = 2048
    assert N % CHUNK == 0

    b2 = b_in.reshape(1, E)

    grid = (L, N // CHUNK, B)
    return pl.pallas_call(
        functools.partial(_embed_body, steps_per_day=288),
        grid=grid,
        in_specs=[
            pl.BlockSpec((1, 1, CHUNK, D), lambda l, c, b: (b, l, c, 0)),
            pl.BlockSpec((D, E), lambda l, c, b: (0, 0)),
            pl.BlockSpec((1, E), lambda l, c, b: (0, 0)),
            pl.BlockSpec(tod_table.shape, lambda l, c, b: (0, 0)),
            pl.BlockSpec(dow_table.shape, lambda l, c, b: (0, 0)),
            pl.BlockSpec((1, CHUNK, A), lambda l, c, b: (l, c, 0)),
        ],
        out_specs=pl.BlockSpec((1, 1, CHUNK, OUT), lambda l, c, b: (b, l, c, 0)),
        out_shape=jax.ShapeDtypeStruct((B, L, N, OUT), jnp.float32),
    )(x, W_in, b2, tod_table, dow_table, adp)


# speedup vs baseline: 3.4365x; 1.0137x over previous
import functools
import jax
import jax.numpy as jnp
from jax import lax
from jax.experimental import pallas as pl
from jax.experimental.pallas import tpu as pltpu


def _embed_body(x_ref, w_ref, b_ref, tab_ref, adp_ref, out_ref, *, steps_per_day, n_tod):
    xv = x_ref[0, 0]
    x1 = xv[:, 1:2]
    x2 = xv[:, 2:3]
    w = w_ref[...]
    xp = (
        xv[:, 0:1] * w[0:1, :]
        + x1 * w[1:2, :]
        + x2 * w[2:3, :]
        + b_ref[...]
    )

    n_rows = tab_ref.shape[0]
    ti = (x1 * jnp.float32(steps_per_day)).astype(jnp.int32)
    di = x2.astype(jnp.int32) + n_tod
    lanes = lax.broadcasted_iota(jnp.int32, (1, n_rows), 1)
    oh = ((ti == lanes) | (di == lanes)).astype(jnp.float32)
    emb = jnp.dot(oh, tab_ref[...], preferred_element_type=jnp.float32)

    out_ref[0, 0] = jnp.concatenate([xp, emb, adp_ref[0]], axis=-1)


def kernel(x, W_in, b_in, tod_table, dow_table, adp):
    B, L, N, D = x.shape
    E = W_in.shape[1]
    A = adp.shape[-1]
    OUT = E * 3 + A
    CHUNK = 2048
    assert N % CHUNK == 0

    b2 = b_in.reshape(1, E)
    n_tod = tod_table.shape[0]
    n_dow = dow_table.shape[0]
    tab = jnp.zeros((n_tod + n_dow, 2 * E), jnp.float32)
    tab = tab.at[:n_tod, :E].set(tod_table).at[n_tod:, E:].set(dow_table)

    grid = (L, N // CHUNK, B)
    return pl.pallas_call(
        functools.partial(_embed_body, steps_per_day=288, n_tod=n_tod),
        grid=grid,
        in_specs=[
            pl.BlockSpec((1, 1, CHUNK, D), lambda l, c, b: (b, l, c, 0)),
            pl.BlockSpec((D, E), lambda l, c, b: (0, 0)),
            pl.BlockSpec((1, E), lambda l, c, b: (0, 0)),
            pl.BlockSpec(tab.shape, lambda l, c, b: (0, 0)),
            pl.BlockSpec((1, CHUNK, A), lambda l, c, b: (l, c, 0)),
        ],
        out_specs=pl.BlockSpec((1, 1, CHUNK, OUT), lambda l, c, b: (b, l, c, 0)),
        out_shape=jax.ShapeDtypeStruct((B, L, N, OUT), jnp.float32),
    )(x, W_in, b2, tab, adp)
